# Initial kernel scaffold; baseline (speedup 1.0000x reference)
import jax
import jax.numpy as jnp
from jax.experimental import pallas as pl


def kernel(w_emb, w1, b1, w2, b2, w3, b3, data_hypo, length_hypo, data_prem, length_prem):
    raise NotImplementedError("write your pallas kernel here")


# speedup vs baseline: 2.8567x; 2.8567x over previous
import functools

import jax
import jax.numpy as jnp
from jax.experimental import pallas as pl
from jax.experimental.pallas import tpu as pltpu

H_PAD = 128


def _round_up(x, m):
    return ((x + m - 1) // m) * m


def _pad2(w, r, c):
    out = jnp.zeros((r, c), jnp.float32)
    return out.at[:w.shape[0], :w.shape[1]].set(w.astype(jnp.float32))


def _nn_body(idx_ref,
             tbl_hbm,
             w1h_ref, w1p_ref, b1_ref,
             w2_ref, b2_ref, w3_ref, b3_ref,
             out_ref,
             tbl_vmem,
             buf,
             cp_sem,
             *, batch_tile, seq_len, grid_y):
    f32 = jnp.float32
    Bt, L = batch_tile, seq_len

    @pl.when(pl.program_id(1) == 0)
    def _():
        cp = pltpu.make_async_copy(tbl_hbm, tbl_vmem, cp_sem)
        cp.start()
        cp.wait()

    tile = pl.program_id(0) * grid_y + pl.program_id(1)
    base = tile * (2 * Bt * L)
    n_pairs = Bt

    def pair_body(bp, carry):
        off = base + bp * (2 * L)
        acc = jnp.zeros((8, 128), f32)
        for t in range(L):
            i0 = pl.multiple_of(idx_ref[off + 2 * t], 2)
            i1 = pl.multiple_of(idx_ref[off + 2 * t + 1], 2)
            s0 = pltpu.bitcast(tbl_vmem[pl.ds(i0, 2), :], jnp.bfloat16)
            s1 = pltpu.bitcast(tbl_vmem[pl.ds(i1, 2), :], jnp.bfloat16)
            both = jnp.concatenate([s0, s1], axis=0)
            acc = acc + both.astype(f32)
        row = pl.multiple_of(bp * 8, 8)
        buf[pl.ds(row, 8), :] = acc
        return carry

    jax.lax.fori_loop(0, n_pairs, pair_body, 0)

    inv_l = f32(1.0 / L)
    z = None
    for j in range(4):
        xh = buf[j:4 * Bt:4, :]
        xp = buf[4 * Bt + j:8 * Bt:4, :]
        d = (jnp.dot(xh, w1h_ref[128 * j:128 * (j + 1), :],
                     preferred_element_type=f32)
             + jnp.dot(xp, w1p_ref[128 * j:128 * (j + 1), :],
                       preferred_element_type=f32))
        z = d if z is None else z + d
    h1 = jnp.maximum(z * inv_l + b1_ref[...], 0.0)
    h2 = jnp.maximum(jnp.dot(h1, w2_ref[...], preferred_element_type=f32)
                     + b2_ref[...], 0.0)
    out_ref[...] = jnp.dot(h2, w3_ref[...], preferred_element_type=f32) + b3_ref[...]


def kernel(w_emb, w1, b1, w2, b2, w3, b3,
           data_hypo, length_hypo, data_prem, length_prem):
    del length_hypo, length_prem
    f32 = jnp.float32

    B, L = data_hypo.shape
    V, D = w_emb.shape
    n_out = w3.shape[1]
    D2 = _round_up(D, 256)
    P = D2 // 256

    Bt = 128 if B >= 256 else max(8, B)
    B_pad = _round_up(B, Bt)
    n_tiles = B_pad // Bt
    gx = 2 if n_tiles % 2 == 0 else 1
    gy = n_tiles // gx

    wb = jnp.pad(w_emb.astype(jnp.bfloat16), ((0, 0), (0, D2 - D)))
    t4 = wb.reshape(V, P, 2, 128).transpose(0, 1, 3, 2)
    tbl = jax.lax.bitcast_convert_type(t4, jnp.int32).reshape(V * P, 128)

    def pad_batch(x):
        x = x.astype(jnp.int32)
        if B_pad == B:
            return x
        return jnp.concatenate([x, jnp.zeros((B_pad - B, L), jnp.int32)], axis=0)

    dh = pad_batch(data_hypo).reshape(n_tiles, Bt, L)
    dp = pad_batch(data_prem).reshape(n_tiles, Bt, L)
    bags = jnp.concatenate([dh, dp], axis=1)
    idx = (bags.reshape(n_tiles, Bt, 2, L).transpose(0, 1, 3, 2)
           .reshape(-1) * P)

    w1h = _pad2(w1[:D], D2, H_PAD)
    w1p = _pad2(w1[D:], D2, H_PAD)
    b1p = _pad2(b1.reshape(1, -1), 1, H_PAD)
    w2p = _pad2(w2, H_PAD, H_PAD)
    b2p = _pad2(b2.reshape(1, -1), 1, H_PAD)
    w3p = _pad2(w3, H_PAD, H_PAD)
    b3p = _pad2(b3.reshape(1, -1), 1, H_PAD)

    body = functools.partial(_nn_body, batch_tile=Bt, seq_len=L, grid_y=gy)

    def full(shape):
        return pl.BlockSpec(shape, lambda i, j, idx_ref: (0,) * len(shape))

    out_pad = pl.pallas_call(
        body,
        out_shape=jax.ShapeDtypeStruct((B_pad, H_PAD), f32),
        grid_spec=pltpu.PrefetchScalarGridSpec(
            num_scalar_prefetch=1,
            grid=(gx, gy),
            in_specs=[
                pl.BlockSpec(memory_space=pl.ANY),
                full((D2, H_PAD)), full((D2, H_PAD)), full((1, H_PAD)),
                full((H_PAD, H_PAD)), full((1, H_PAD)),
                full((H_PAD, H_PAD)), full((1, H_PAD)),
            ],
            out_specs=pl.BlockSpec((Bt, H_PAD),
                                   lambda i, j, idx_ref: (i * gy + j, 0)),
            scratch_shapes=[
                pltpu.VMEM((V * P, 128), jnp.int32),
                pltpu.VMEM((8 * Bt, 128), f32),
                pltpu.SemaphoreType.DMA,
            ]),
        compiler_params=pltpu.CompilerParams(
            dimension_semantics=("parallel", "arbitrary"),
            vmem_limit_bytes=63 * 1024 * 1024,
        ),
    )(idx, tbl, w1h, w1p, b1p, w2p, b2p, w3p, b3p)

    return out_pad[:B, :n_out]
